# baseline (speedup 1.0000x reference)
import functools

import jax
import jax.numpy as jnp
from jax import lax
from jax.experimental import pallas as pl
from jax.experimental.pallas import tpu as pltpu
from jax.experimental.pallas import tpu_sc as plsc

D_MODEL = 256
MAX_LEN = 2048
SEQ_LEN = 512
WIN_START = MAX_LEN - SEQ_LEN
WIN_ROWS = 2 * SEQ_LEN
ROW_ELEMS = SEQ_LEN * D_MODEL

NUM_CORES = 2
NUM_SUBCORES = 16
NUM_WORKERS = NUM_CORES * NUM_SUBCORES
ROWS_PER_WORKER = SEQ_LEN // NUM_WORKERS


def _body(table_hbm, out_hbm, win, sem):
    cid = lax.axis_index("c")
    sid = lax.axis_index("s")

    @pl.when(sid == 0)
    def _load():
        pltpu.sync_copy(
            table_hbm.at[pl.ds(WIN_START * D_MODEL, WIN_ROWS * D_MODEL)], win)

    plsc.subcore_barrier()

    wid = sid * NUM_CORES + cid
    base = wid * ROWS_PER_WORKER

    handles = []
    for k in range(ROWS_PER_WORKER):
        i = base + k
        src = pl.multiple_of(((SEQ_LEN - 1) - i) * D_MODEL, D_MODEL)
        dst = pl.multiple_of(i * ROW_ELEMS, ROW_ELEMS)
        handles.append(pltpu.async_copy(win.at[pl.ds(src, ROW_ELEMS)],
                                        out_hbm.at[pl.ds(dst, ROW_ELEMS)], sem))
    for h in handles:
        h.wait()


def kernel(seq_len, table):
    del seq_len
    mesh = plsc.VectorSubcoreMesh(core_axis_name="c", subcore_axis_name="s")
    run = functools.partial(
        pl.kernel,
        mesh=mesh,
        out_type=jax.ShapeDtypeStruct((SEQ_LEN * SEQ_LEN * D_MODEL,), jnp.float32),
        scratch_types=[
            pltpu.VMEM_SHARED((WIN_ROWS * D_MODEL,), jnp.float32),
            pltpu.SemaphoreType.DMA,
        ],
    )(_body)
    flat = run(table.reshape(-1))
    return flat.reshape(SEQ_LEN, SEQ_LEN, D_MODEL)


# speedup vs baseline: 2.5277x; 2.5277x over previous
import functools

import jax
import jax.numpy as jnp
from jax import lax
from jax.experimental import pallas as pl
from jax.experimental.pallas import tpu as pltpu
from jax.experimental.pallas import tpu_sc as plsc

D_MODEL = 256
MAX_LEN = 2048
SEQ_LEN = 512
WIN_START = MAX_LEN - SEQ_LEN
NUM_SHIFTS = 8
WIN_ROWS = 1016

NUM_CORES = 2
NUM_SUBCORES = 16
NUM_WORKERS = NUM_CORES * NUM_SUBCORES
ROWS_PER_WORKER = SEQ_LEN // NUM_WORKERS


def _body(win_hbm, out_hbm, win, sem):
    cid = lax.axis_index("c")
    sid = lax.axis_index("s")

    @pl.when(sid == 0)
    def _load():
        pltpu.sync_copy(win_hbm, win)

    plsc.subcore_barrier()

    wid = sid * NUM_CORES + cid
    base = wid * ROWS_PER_WORKER

    handles = []
    for k in range(ROWS_PER_WORKER):
        i = base + k
        start = (SEQ_LEN - 1) - i
        s = lax.rem(start, NUM_SHIFTS)
        a = pl.multiple_of(start - s, NUM_SHIFTS)
        handles.append(pltpu.async_copy(
            win.at[s, pl.ds(a, SEQ_LEN)], out_hbm.at[i], sem))
    for h in handles:
        h.wait()


def kernel(seq_len, table):
    del seq_len
    win8 = jnp.stack(
        [lax.slice_in_dim(table, WIN_START + s, WIN_START + s + WIN_ROWS, axis=0)
         for s in range(NUM_SHIFTS)])
    mesh = plsc.VectorSubcoreMesh(core_axis_name="c", subcore_axis_name="s")
    run = functools.partial(
        pl.kernel,
        mesh=mesh,
        out_type=jax.ShapeDtypeStruct((SEQ_LEN, SEQ_LEN, D_MODEL), jnp.float32),
        scratch_types=[
            pltpu.VMEM_SHARED((NUM_SHIFTS, WIN_ROWS, D_MODEL), jnp.float32),
            pltpu.SemaphoreType.DMA,
        ],
    )(_body)
    return run(win8)


# speedup vs baseline: 2.5952x; 1.0267x over previous
import functools

import jax
import jax.numpy as jnp
from jax import lax
from jax.experimental import pallas as pl
from jax.experimental.pallas import tpu as pltpu
from jax.experimental.pallas import tpu_sc as plsc

D_MODEL = 256
MAX_LEN = 2048
SEQ_LEN = 512
WIN_START = MAX_LEN - SEQ_LEN
NUM_SHIFTS = 8
WIN_ROWS = 1016

NUM_CORES = 2
NUM_SUBCORES = 16
SHIFTS_PER_CORE = NUM_SHIFTS // NUM_CORES
ABLOCK = SEQ_LEN // NUM_SHIFTS // NUM_SUBCORES


def _body(win_hbm, out_hbm, win, sem):
    cid = lax.axis_index("c")
    sid = lax.axis_index("s")

    @pl.when(sid == 0)
    def _load():
        pltpu.sync_copy(win_hbm.at[pl.ds(SHIFTS_PER_CORE * cid,
                                         SHIFTS_PER_CORE)], win)

    plsc.subcore_barrier()

    handles = []
    for ka in range(ABLOCK):
        a = (sid * ABLOCK + ka) * NUM_SHIFTS
        for t in range(SHIFTS_PER_CORE):
            start = a + SHIFTS_PER_CORE * cid + t
            i = (SEQ_LEN - 1) - start
            handles.append(pltpu.async_copy(
                win.at[t, pl.ds(pl.multiple_of(a, NUM_SHIFTS), SEQ_LEN)],
                out_hbm.at[i], sem))
    for h in handles:
        h.wait()


def kernel(seq_len, table):
    del seq_len
    win8 = jnp.stack(
        [lax.slice_in_dim(table, WIN_START + s, WIN_START + s + WIN_ROWS, axis=0)
         for s in range(NUM_SHIFTS)])
    mesh = plsc.VectorSubcoreMesh(core_axis_name="c", subcore_axis_name="s")
    run = functools.partial(
        pl.kernel,
        mesh=mesh,
        out_type=jax.ShapeDtypeStruct((SEQ_LEN, SEQ_LEN, D_MODEL), jnp.float32),
        scratch_types=[
            pltpu.VMEM_SHARED((SHIFTS_PER_CORE, WIN_ROWS, D_MODEL), jnp.float32),
            pltpu.SemaphoreType.DMA,
        ],
    )(_body)
    return run(win8)


# speedup vs baseline: 2.6001x; 1.0019x over previous
import functools

import jax
import jax.numpy as jnp
from jax import lax
from jax.experimental import pallas as pl
from jax.experimental.pallas import tpu as pltpu
from jax.experimental.pallas import tpu_sc as plsc

D_MODEL = 256
MAX_LEN = 2048
SEQ_LEN = 512
WIN_START = MAX_LEN - SEQ_LEN
NUM_SHIFTS = 8
WIN_ROWS = 1016

NUM_CORES = 2
NUM_SUBCORES = 16
SHIFTS_PER_CORE = NUM_SHIFTS // NUM_CORES
ABLOCK = SEQ_LEN // NUM_SHIFTS // NUM_SUBCORES


def _body(win_hbm, out_hbm, win, sem):
    cid = lax.axis_index("c")
    sid = lax.axis_index("s")

    for t in range(SHIFTS_PER_CORE):
        for c in range(4):
            rows = min(256, WIN_ROWS - c * 256)

            @pl.when(sid == t * 4 + c)
            def _load(t=t, c=c, rows=rows):
                pltpu.sync_copy(
                    win_hbm.at[SHIFTS_PER_CORE * cid + t, pl.ds(c * 256, rows)],
                    win.at[t, pl.ds(c * 256, rows)])

    plsc.subcore_barrier()

    handles = []
    for ka in range(ABLOCK):
        a = (sid * ABLOCK + ka) * NUM_SHIFTS
        for t in range(SHIFTS_PER_CORE):
            start = a + SHIFTS_PER_CORE * cid + t
            i = (SEQ_LEN - 1) - start
            handles.append(pltpu.async_copy(
                win.at[t, pl.ds(pl.multiple_of(a, NUM_SHIFTS), SEQ_LEN)],
                out_hbm.at[i], sem))
    for h in handles:
        h.wait()


def kernel(seq_len, table):
    del seq_len
    win8 = jnp.stack(
        [lax.slice_in_dim(table, WIN_START + s, WIN_START + s + WIN_ROWS, axis=0)
         for s in range(NUM_SHIFTS)])
    mesh = plsc.VectorSubcoreMesh(core_axis_name="c", subcore_axis_name="s")
    run = functools.partial(
        pl.kernel,
        mesh=mesh,
        out_type=jax.ShapeDtypeStruct((SEQ_LEN, SEQ_LEN, D_MODEL), jnp.float32),
        scratch_types=[
            pltpu.VMEM_SHARED((SHIFTS_PER_CORE, WIN_ROWS, D_MODEL), jnp.float32),
            pltpu.SemaphoreType.DMA,
        ],
    )(_body)
    return run(win8)


# speedup vs baseline: 2.7279x; 1.0492x over previous
import functools

import jax
import jax.numpy as jnp
from jax import lax
from jax.experimental import pallas as pl
from jax.experimental.pallas import tpu as pltpu
from jax.experimental.pallas import tpu_sc as plsc

D_MODEL = 256
MAX_LEN = 2048
SEQ_LEN = 512
WIN_START = MAX_LEN - SEQ_LEN
NUM_SHIFTS = 8
WIN_ROWS = 1016

NUM_CORES = 2
NUM_SUBCORES = 16
SHIFTS_PER_CORE = NUM_SHIFTS // NUM_CORES
ABLOCK = SEQ_LEN // NUM_SHIFTS // NUM_SUBCORES


def _body(win_hbm, out_hbm, win, sem):
    cid = lax.axis_index("c")
    sid = lax.axis_index("s")

    for t in range(SHIFTS_PER_CORE):
        for c in range(4):
            rows = min(256, WIN_ROWS - c * 256)

            @pl.when(sid == t * 4 + c)
            def _load(t=t, c=c, rows=rows):
                pltpu.sync_copy(
                    win_hbm.at[SHIFTS_PER_CORE * cid + t, pl.ds(c * 256, rows)],
                    win.at[t, pl.ds(c * 256, rows)])

    plsc.subcore_barrier()

    handles = []
    for ka in range(ABLOCK):
        a = (sid * ABLOCK + ka) * NUM_SHIFTS
        for t in range(SHIFTS_PER_CORE):
            start = a + SHIFTS_PER_CORE * cid + t
            i = (SEQ_LEN - 1) - start
            handles.append(pltpu.async_copy(
                win.at[t, pl.ds(pl.multiple_of(a, NUM_SHIFTS), SEQ_LEN)],
                out_hbm.at[i], sem))
    for h in handles:
        h.wait()


def _prep_body(table_ref, win8_ref):
    for s in range(NUM_SHIFTS):
        win8_ref[s] = table_ref[pl.ds(WIN_START + s, WIN_ROWS), :]


def kernel(seq_len, table):
    del seq_len
    win8 = pl.pallas_call(
        _prep_body,
        out_shape=jax.ShapeDtypeStruct((NUM_SHIFTS, WIN_ROWS, D_MODEL),
                                       jnp.float32),
    )(table)
    mesh = plsc.VectorSubcoreMesh(core_axis_name="c", subcore_axis_name="s")
    run = functools.partial(
        pl.kernel,
        mesh=mesh,
        out_type=jax.ShapeDtypeStruct((SEQ_LEN, SEQ_LEN, D_MODEL), jnp.float32),
        scratch_types=[
            pltpu.VMEM_SHARED((SHIFTS_PER_CORE, WIN_ROWS, D_MODEL), jnp.float32),
            pltpu.SemaphoreType.DMA,
        ],
    )(_body)
    return run(win8)


# speedup vs baseline: 2.8835x; 1.0570x over previous
import functools

import jax
import jax.numpy as jnp
from jax import lax
from jax.experimental import pallas as pl
from jax.experimental.pallas import tpu as pltpu
from jax.experimental.pallas import tpu_sc as plsc

D_MODEL = 256
MAX_LEN = 2048
SEQ_LEN = 512
WIN_START = MAX_LEN - SEQ_LEN
NUM_SHIFTS = 8
WIN_ROWS = 1016

NUM_CORES = 2
NUM_SUBCORES = 16

TC_ROWS = 128
SC_ROWS = SEQ_LEN - TC_ROWS
SHIFTS_PER_CORE = NUM_SHIFTS // NUM_CORES
ABLOCK = SC_ROWS // NUM_SHIFTS // NUM_SUBCORES
TC_BLOCK = 8


def _sc_body(win_hbm, out_hbm, win, sem):
    cid = lax.axis_index("c")
    sid = lax.axis_index("s")

    for t in range(SHIFTS_PER_CORE):
        for c in range(4):
            rows = min(256, WIN_ROWS - c * 256)

            @pl.when(sid == t * 4 + c)
            def _load(t=t, c=c, rows=rows):
                pltpu.sync_copy(
                    win_hbm.at[SHIFTS_PER_CORE * cid + t, pl.ds(c * 256, rows)],
                    win.at[t, pl.ds(c * 256, rows)])

    plsc.subcore_barrier()

    handles = []
    for ka in range(ABLOCK):
        a = (sid * ABLOCK + ka) * NUM_SHIFTS
        for t in range(SHIFTS_PER_CORE):
            start = a + SHIFTS_PER_CORE * cid + t
            i = (SEQ_LEN - 1) - start
            handles.append(pltpu.async_copy(
                win.at[t, pl.ds(pl.multiple_of(a, NUM_SHIFTS), SEQ_LEN)],
                out_hbm.at[i], sem))
    for h in handles:
        h.wait()


def _prep_body(table_ref, win8_ref):
    for s in range(NUM_SHIFTS):
        win8_ref[s] = table_ref[pl.ds(WIN_START + s, WIN_ROWS), :]


def _tc_finish_body(win8_ref, _aliased_ref, out_ref):
    g = pl.program_id(0)
    for k in range(TC_BLOCK):
        i = g * TC_BLOCK + k
        s = (NUM_SHIFTS - 1 - k) % NUM_SHIFTS
        a = pl.multiple_of((SEQ_LEN - 1) - i - s, NUM_SHIFTS)
        out_ref[k] = win8_ref[s, pl.ds(a, SEQ_LEN), :]


def kernel(seq_len, table):
    del seq_len
    win8 = pl.pallas_call(
        _prep_body,
        out_shape=jax.ShapeDtypeStruct((NUM_SHIFTS, WIN_ROWS, D_MODEL),
                                       jnp.float32),
    )(table)

    mesh = plsc.VectorSubcoreMesh(core_axis_name="c", subcore_axis_name="s")
    sc_run = functools.partial(
        pl.kernel,
        mesh=mesh,
        out_type=jax.ShapeDtypeStruct((SEQ_LEN, SEQ_LEN, D_MODEL), jnp.float32),
        scratch_types=[
            pltpu.VMEM_SHARED((SHIFTS_PER_CORE, WIN_ROWS, D_MODEL), jnp.float32),
            pltpu.SemaphoreType.DMA,
        ],
    )(_sc_body)
    partial_out = sc_run(win8)

    return pl.pallas_call(
        _tc_finish_body,
        grid=(TC_ROWS // TC_BLOCK,),
        in_specs=[
            pl.BlockSpec((NUM_SHIFTS, WIN_ROWS, D_MODEL), lambda g: (0, 0, 0)),
            pl.BlockSpec(memory_space=pltpu.HBM),
        ],
        out_specs=pl.BlockSpec((TC_BLOCK, SEQ_LEN, D_MODEL),
                               lambda g: (g, 0, 0)),
        out_shape=jax.ShapeDtypeStruct((SEQ_LEN, SEQ_LEN, D_MODEL),
                                       jnp.float32),
        input_output_aliases={1: 0},
    )(win8, partial_out)


# speedup vs baseline: 3.1567x; 1.0947x over previous
import functools

import jax
import jax.numpy as jnp
from jax import lax
from jax.experimental import pallas as pl
from jax.experimental.pallas import tpu as pltpu
from jax.experimental.pallas import tpu_sc as plsc

D_MODEL = 256
MAX_LEN = 2048
SEQ_LEN = 512
WIN_START = MAX_LEN - SEQ_LEN
NUM_SHIFTS = 8
WIN_ROWS = 1016

NUM_CORES = 2
NUM_SUBCORES = 16

TC_ROWS = 256
SC_ROWS = SEQ_LEN - TC_ROWS
SHIFTS_PER_CORE = NUM_SHIFTS // NUM_CORES
ABLOCK = SC_ROWS // NUM_SHIFTS // NUM_SUBCORES
TC_BLOCK = 8


def _sc_body(win_hbm, out_hbm, win, sem):
    cid = lax.axis_index("c")
    sid = lax.axis_index("s")

    for t in range(SHIFTS_PER_CORE):
        for c in range(4):
            rows = min(256, WIN_ROWS - c * 256)

            @pl.when(sid == t * 4 + c)
            def _load(t=t, c=c, rows=rows):
                pltpu.sync_copy(
                    win_hbm.at[SHIFTS_PER_CORE * cid + t, pl.ds(c * 256, rows)],
                    win.at[t, pl.ds(c * 256, rows)])

    plsc.subcore_barrier()

    handles = []
    for ka in range(ABLOCK):
        a = (sid * ABLOCK + ka) * NUM_SHIFTS
        for t in range(SHIFTS_PER_CORE):
            start = a + SHIFTS_PER_CORE * cid + t
            i = (SEQ_LEN - 1) - start
            handles.append(pltpu.async_copy(
                win.at[t, pl.ds(pl.multiple_of(a, NUM_SHIFTS), SEQ_LEN)],
                out_hbm.at[i], sem))
    for h in handles:
        h.wait()


def _prep_body(table_ref, win8_ref):
    for s in range(NUM_SHIFTS):
        win8_ref[s] = table_ref[pl.ds(WIN_START + s, WIN_ROWS), :]


def _tc_finish_body(win8_ref, _aliased_ref, out_ref):
    g = pl.program_id(0)
    for k in range(TC_BLOCK):
        i = g * TC_BLOCK + k
        s = (NUM_SHIFTS - 1 - k) % NUM_SHIFTS
        a = pl.multiple_of((SEQ_LEN - 1) - i - s, NUM_SHIFTS)
        out_ref[k] = win8_ref[s, pl.ds(a, SEQ_LEN), :]


def kernel(seq_len, table):
    del seq_len
    win8 = pl.pallas_call(
        _prep_body,
        out_shape=jax.ShapeDtypeStruct((NUM_SHIFTS, WIN_ROWS, D_MODEL),
                                       jnp.float32),
    )(table)

    mesh = plsc.VectorSubcoreMesh(core_axis_name="c", subcore_axis_name="s")
    sc_run = functools.partial(
        pl.kernel,
        mesh=mesh,
        out_type=jax.ShapeDtypeStruct((SEQ_LEN, SEQ_LEN, D_MODEL), jnp.float32),
        scratch_types=[
            pltpu.VMEM_SHARED((SHIFTS_PER_CORE, WIN_ROWS, D_MODEL), jnp.float32),
            pltpu.SemaphoreType.DMA,
        ],
    )(_sc_body)
    partial_out = sc_run(win8)

    return pl.pallas_call(
        _tc_finish_body,
        grid=(TC_ROWS // TC_BLOCK,),
        in_specs=[
            pl.BlockSpec((NUM_SHIFTS, WIN_ROWS, D_MODEL), lambda g: (0, 0, 0)),
            pl.BlockSpec(memory_space=pltpu.HBM),
        ],
        out_specs=pl.BlockSpec((TC_BLOCK, SEQ_LEN, D_MODEL),
                               lambda g: (g, 0, 0)),
        out_shape=jax.ShapeDtypeStruct((SEQ_LEN, SEQ_LEN, D_MODEL),
                                       jnp.float32),
        input_output_aliases={1: 0},
    )(win8, partial_out)


# speedup vs baseline: 3.1896x; 1.0104x over previous
import functools

import jax
import jax.numpy as jnp
from jax import lax
from jax.experimental import pallas as pl
from jax.experimental.pallas import tpu as pltpu
from jax.experimental.pallas import tpu_sc as plsc

D_MODEL = 256
MAX_LEN = 2048
SEQ_LEN = 512
WIN_START = MAX_LEN - SEQ_LEN
NUM_SHIFTS = 8
WIN_ROWS = 1016

NUM_CORES = 2
NUM_SUBCORES = 16

TC_ROWS = 256
SC_ROWS = SEQ_LEN - TC_ROWS
SHIFTS_PER_CORE = NUM_SHIFTS // NUM_CORES
ABLOCK = SC_ROWS // NUM_SHIFTS // NUM_SUBCORES
TC_BLOCK = 8
SC_WIN_ROWS = SC_ROWS - NUM_SHIFTS + SEQ_LEN


def _sc_body(win_hbm, out_hbm, win, sem):
    cid = lax.axis_index("c")
    sid = lax.axis_index("s")

    for t in range(SHIFTS_PER_CORE):
        for c in range(4):
            rows = min(192, SC_WIN_ROWS - c * 192)

            @pl.when(sid == t * 4 + c)
            def _load(t=t, c=c, rows=rows):
                pltpu.sync_copy(
                    win_hbm.at[SHIFTS_PER_CORE * cid + t, pl.ds(c * 192, rows)],
                    win.at[t, pl.ds(c * 192, rows)])

    plsc.subcore_barrier()

    handles = []
    for ka in range(ABLOCK):
        a = (sid * ABLOCK + ka) * NUM_SHIFTS
        for t in range(SHIFTS_PER_CORE):
            start = a + SHIFTS_PER_CORE * cid + t
            i = (SEQ_LEN - 1) - start
            handles.append(pltpu.async_copy(
                win.at[t, pl.ds(pl.multiple_of(a, NUM_SHIFTS), SEQ_LEN)],
                out_hbm.at[i], sem))
    for h in handles:
        h.wait()


def _prep_body(table_ref, win8_ref):
    for s in range(NUM_SHIFTS):
        win8_ref[s] = table_ref[pl.ds(WIN_START + s, WIN_ROWS), :]


def _tc_finish_body(win8_ref, _aliased_ref, out_ref):
    g = pl.program_id(0)
    for k in range(TC_BLOCK):
        i = g * TC_BLOCK + k
        s = (NUM_SHIFTS - 1 - k) % NUM_SHIFTS
        a = pl.multiple_of((SEQ_LEN - 1) - i - s, NUM_SHIFTS)
        out_ref[k] = win8_ref[s, pl.ds(a, SEQ_LEN), :]


def kernel(seq_len, table):
    del seq_len
    win8 = pl.pallas_call(
        _prep_body,
        out_shape=jax.ShapeDtypeStruct((NUM_SHIFTS, WIN_ROWS, D_MODEL),
                                       jnp.float32),
    )(table)

    mesh = plsc.VectorSubcoreMesh(core_axis_name="c", subcore_axis_name="s")
    sc_run = functools.partial(
        pl.kernel,
        mesh=mesh,
        out_type=jax.ShapeDtypeStruct((SEQ_LEN, SEQ_LEN, D_MODEL), jnp.float32),
        scratch_types=[
            pltpu.VMEM_SHARED((SHIFTS_PER_CORE, SC_WIN_ROWS, D_MODEL),
                              jnp.float32),
            pltpu.SemaphoreType.DMA,
        ],
    )(_sc_body)
    partial_out = sc_run(win8)

    return pl.pallas_call(
        _tc_finish_body,
        grid=(TC_ROWS // TC_BLOCK,),
        in_specs=[
            pl.BlockSpec((NUM_SHIFTS, WIN_ROWS, D_MODEL), lambda g: (0, 0, 0)),
            pl.BlockSpec(memory_space=pltpu.HBM),
        ],
        out_specs=pl.BlockSpec((TC_BLOCK, SEQ_LEN, D_MODEL),
                               lambda g: (g, 0, 0)),
        out_shape=jax.ShapeDtypeStruct((SEQ_LEN, SEQ_LEN, D_MODEL),
                                       jnp.float32),
        input_output_aliases={1: 0},
    )(win8, partial_out)


# speedup vs baseline: 3.2100x; 1.0064x over previous
import functools

import jax
import jax.numpy as jnp
from jax import lax
from jax.experimental import pallas as pl
from jax.experimental.pallas import tpu as pltpu
from jax.experimental.pallas import tpu_sc as plsc

D_MODEL = 256
MAX_LEN = 2048
SEQ_LEN = 512
WIN_START = MAX_LEN - SEQ_LEN
NUM_SHIFTS = 8
WIN_ROWS = 1016

NUM_CORES = 2
NUM_SUBCORES = 16

TC_ROWS = 256
SC_ROWS = SEQ_LEN - TC_ROWS
SHIFTS_PER_CORE = NUM_SHIFTS // NUM_CORES
ABLOCK = SC_ROWS // NUM_SHIFTS // NUM_SUBCORES
TC_BLOCK = 8
SC_WIN_ROWS = SC_ROWS - NUM_SHIFTS + SEQ_LEN


def _sc_body(win_hbm, out_hbm, win, sem):
    cid = lax.axis_index("c")
    sid = lax.axis_index("s")

    for t in range(SHIFTS_PER_CORE):
        for c in range(4):
            rows = min(192, SC_WIN_ROWS - c * 192)

            @pl.when(sid == t * 4 + c)
            def _load(t=t, c=c, rows=rows):
                pltpu.sync_copy(
                    win_hbm.at[SHIFTS_PER_CORE * cid + t, pl.ds(c * 192, rows)],
                    win.at[t, pl.ds(c * 192, rows)])

    plsc.subcore_barrier()

    handles = []
    for ka in range(ABLOCK):
        a = (sid * ABLOCK + ka) * NUM_SHIFTS
        for t in range(SHIFTS_PER_CORE):
            start = a + SHIFTS_PER_CORE * cid + t
            i = (SEQ_LEN - 1) - start
            handles.append(pltpu.async_copy(
                win.at[t, pl.ds(pl.multiple_of(a, NUM_SHIFTS), SEQ_LEN)],
                out_hbm.at[i], sem))
    for h in handles:
        h.wait()


def _prep_body(a_ref, b_ref, win8_ref):
    for s in range(NUM_SHIFTS):
        win8_ref[s, : SEQ_LEN - s, :] = a_ref[pl.ds(s, SEQ_LEN - s), :]
        win8_ref[s, SEQ_LEN - s :, :] = b_ref[pl.ds(0, WIN_ROWS - SEQ_LEN + s), :]


def _tc_finish_body(win8_ref, _aliased_ref, out_ref):
    g = pl.program_id(0)
    for k in range(TC_BLOCK):
        i = g * TC_BLOCK + k
        s = (NUM_SHIFTS - 1 - k) % NUM_SHIFTS
        a = pl.multiple_of((SEQ_LEN - 1) - i - s, NUM_SHIFTS)
        out_ref[k] = win8_ref[s, pl.ds(a, SEQ_LEN), :]


def kernel(seq_len, table):
    del seq_len
    win8 = pl.pallas_call(
        _prep_body,
        grid=(1,),
        in_specs=[
            pl.BlockSpec((SEQ_LEN, D_MODEL),
                         lambda g: (WIN_START // SEQ_LEN, 0)),
            pl.BlockSpec((SEQ_LEN, D_MODEL),
                         lambda g: (WIN_START // SEQ_LEN + 1, 0)),
        ],
        out_specs=pl.BlockSpec((NUM_SHIFTS, WIN_ROWS, D_MODEL),
                               lambda g: (0, 0, 0)),
        out_shape=jax.ShapeDtypeStruct((NUM_SHIFTS, WIN_ROWS, D_MODEL),
                                       jnp.float32),
    )(table, table)

    mesh = plsc.VectorSubcoreMesh(core_axis_name="c", subcore_axis_name="s")
    sc_run = functools.partial(
        pl.kernel,
        mesh=mesh,
        out_type=jax.ShapeDtypeStruct((SEQ_LEN, SEQ_LEN, D_MODEL), jnp.float32),
        scratch_types=[
            pltpu.VMEM_SHARED((SHIFTS_PER_CORE, SC_WIN_ROWS, D_MODEL),
                              jnp.float32),
            pltpu.SemaphoreType.DMA,
        ],
    )(_sc_body)
    partial_out = sc_run(win8)

    return pl.pallas_call(
        _tc_finish_body,
        grid=(TC_ROWS // TC_BLOCK,),
        in_specs=[
            pl.BlockSpec((NUM_SHIFTS, WIN_ROWS, D_MODEL), lambda g: (0, 0, 0)),
            pl.BlockSpec(memory_space=pltpu.HBM),
        ],
        out_specs=pl.BlockSpec((TC_BLOCK, SEQ_LEN, D_MODEL),
                               lambda g: (g, 0, 0)),
        out_shape=jax.ShapeDtypeStruct((SEQ_LEN, SEQ_LEN, D_MODEL),
                                       jnp.float32),
        input_output_aliases={1: 0},
    )(win8, partial_out)
